# Initial kernel scaffold; baseline (speedup 1.0000x reference)
import jax
import jax.numpy as jnp
from jax.experimental import pallas as pl


def kernel(x0_logprob, uniform_noise, xt, timestep, batch):
    raise NotImplementedError("write your pallas kernel here")


# speedup vs baseline: 10.7399x; 10.7399x over previous
import functools

import jax
import jax.numpy as jnp
import numpy as np
from jax.experimental import pallas as pl
from jax.experimental.pallas import tpu as pltpu

_N = 32768
_NCLASS = 128
_B = 128
_T = 50
_EPS = 1e-30

_betas = np.array([0.0004 * (k + 1) for k in range(_T)], dtype=np.float64)
_alphas = 1.0 - _betas
_alpha_bars = np.cumprod(_alphas, axis=0)
_prev_alpha_bars = np.concatenate([[1.0], _alpha_bars[:-1]])
_LA = np.log(_alphas + _EPS).astype(np.float32)
_L1MA = np.log(1.0 - _alphas + _EPS).astype(np.float32)
_LPAB = np.log(_prev_alpha_bars + _EPS).astype(np.float32)
_L1MPAB = np.log(1.0 - _prev_alpha_bars + _EPS).astype(np.float32)
_C_INIT = np.float32(np.maximum(np.log(1.0 / _NCLASS + _EPS), -30.0))
_LOG_EPS30 = np.log(np.float32(_EPS))


_TABLE = np.zeros((8, 64), dtype=np.float32)
_TABLE[0, :_T] = _LA
_TABLE[1, :_T] = (_LOG_EPS30 + _LA).astype(np.float32)
_TABLE[2, :_T] = (_L1MA + _C_INIT).astype(np.float32)
_TABLE[3, :_T] = _LPAB
_TABLE[4, :_T] = (_L1MPAB + _C_INIT).astype(np.float32)
_TABLE[5, 0] = 1.0


def _body(x0_ref, u_ref, xt_ref, b_ref, ts_ref, tab_ref, lp_ref, s_ref):
    x0 = x0_ref[...]
    r = x0.shape[0]
    zero = jnp.float32(0.0)
    lane = jax.lax.broadcasted_iota(jnp.int32, (1, _NCLASS), 1)
    b = b_ref[...]
    oh_b = b == lane
    ts_f = ts_ref[...].astype(jnp.float32)
    t = jnp.sum(jnp.where(oh_b, ts_f, zero), axis=1, keepdims=True)
    lane64 = jax.lax.broadcasted_iota(jnp.int32, (1, 64), 1)
    oh_t = t.astype(jnp.int32) == lane64
    tab = tab_ref[...]

    def _pick(j):
        return jnp.sum(jnp.where(oh_t, tab[j:j + 1, :], zero),
                       axis=1, keepdims=True)

    la = _pick(0)
    la_eps = _pick(1)
    l1ma_c = _pick(2)
    lpab = _pick(3)
    r2 = _pick(4)
    tzm = _pick(5) > 0.5

    xtm = xt_ref[...] == lane
    al = jnp.where(xtm, la, la_eps)
    ml = jnp.maximum(al, l1ma_c)
    left = ml + jnp.log(jnp.exp(al - ml) + jnp.exp(l1ma_c - ml))
    a = lpab + x0
    m = jnp.maximum(a, r2)
    right = m + jnp.log(jnp.exp(a - m) + jnp.exp(r2 - m))
    lp = left + right
    rowmax = jnp.max(lp, axis=1, keepdims=True)
    lse = rowmax + jnp.log(jnp.sum(jnp.exp(lp - rowmax), axis=1, keepdims=True))
    lp_out = jnp.where(tzm, x0, lp - lse)
    lp_ref[...] = lp_out

    g = -jnp.log(-jnp.log(u_ref[...] + jnp.float32(_EPS)) + jnp.float32(_EPS))
    score = g + lp_out
    lane_i = jax.lax.broadcasted_iota(jnp.int32, (r, _NCLASS), 1)
    smax = jnp.max(score, axis=1, keepdims=True)
    sidx = jnp.min(jnp.where(score == smax, lane_i, _NCLASS), axis=1, keepdims=True)
    xmax = jnp.max(x0, axis=1, keepdims=True)
    xidx = jnp.min(jnp.where(x0 == xmax, lane_i, _NCLASS), axis=1, keepdims=True)
    s_ref[...] = jnp.where(tzm, xidx, sidx)


def kernel(x0_logprob, uniform_noise, xt, timestep, batch):
    r = 512
    grid = (_N // r,)
    xt2 = xt.reshape(_N, 1)
    b2 = batch.reshape(_N, 1)
    ts2 = timestep.reshape(1, _B)
    tab = jnp.asarray(_TABLE)
    lp, s2 = pl.pallas_call(
        _body,
        grid=grid,
        in_specs=[
            pl.BlockSpec((r, _NCLASS), lambda i: (i, 0)),
            pl.BlockSpec((r, _NCLASS), lambda i: (i, 0)),
            pl.BlockSpec((r, 1), lambda i: (i, 0)),
            pl.BlockSpec((r, 1), lambda i: (i, 0)),
            pl.BlockSpec((1, _B), lambda i: (0, 0)),
            pl.BlockSpec((8, 64), lambda i: (0, 0)),
        ],
        out_specs=[
            pl.BlockSpec((r, _NCLASS), lambda i: (i, 0)),
            pl.BlockSpec((r, 1), lambda i: (i, 0)),
        ],
        out_shape=[
            jax.ShapeDtypeStruct((_N, _NCLASS), jnp.float32),
            jax.ShapeDtypeStruct((_N, 1), jnp.int32),
        ],
        compiler_params=pltpu.CompilerParams(dimension_semantics=("parallel",)),
    )(x0_logprob, uniform_noise, xt2, b2, ts2, tab)
    return lp, s2.reshape(_N)
